# baseline (speedup 1.0000x reference)
import jax
import jax.numpy as jnp
from jax.experimental import pallas as pl
from jax.experimental.pallas import tpu as pltpu


def _add_body(x_ref, rt_ref, ft_ref, o_ref):
    rt = rt_ref[...]
    ft = ft_ref[...]
    rank_emb = jnp.broadcast_to(rt[:, None, :], (8, 8, 64)).reshape(64, 64)
    file_emb = jnp.broadcast_to(ft[None, :, :], (8, 8, 64)).reshape(64, 64)
    x = x_ref[...]
    o_ref[:, :, 0:64] = x[:, :, 0:64] + rank_emb[None]
    o_ref[:, :, 64:128] = x[:, :, 64:128] + file_emb[None]


def kernel(x, rank_table, file_table):
    B, S, D = x.shape
    BB = 128
    grid = (B // BB,)
    return pl.pallas_call(
        _add_body,
        grid=grid,
        in_specs=[
            pl.BlockSpec((BB, S, D), lambda i: (i, 0, 0)),
            pl.BlockSpec((8, 64), lambda i: (0, 0)),
            pl.BlockSpec((8, 64), lambda i: (0, 0)),
        ],
        out_specs=pl.BlockSpec((BB, S, D), lambda i: (i, 0, 0)),
        out_shape=jax.ShapeDtypeStruct((B, S, D), x.dtype),
    )(x, rank_table, file_table)


# speedup vs baseline: 1.0235x; 1.0235x over previous
import jax
import jax.numpy as jnp
from jax.experimental import pallas as pl
from jax.experimental.pallas import tpu as pltpu


def _add_body(x_ref, rt_ref, ft_ref, o_ref):
    rt = rt_ref[...]
    ft = ft_ref[...]
    rank_emb = jnp.broadcast_to(rt[:, None, :], (8, 8, 64)).reshape(64, 64)
    file_emb = jnp.broadcast_to(ft[None, :, :], (8, 8, 64)).reshape(64, 64)
    x = x_ref[...]
    o_ref[:, :, 0:64] = x[:, :, 0:64] + rank_emb[None]
    o_ref[:, :, 64:128] = x[:, :, 64:128] + file_emb[None]


def kernel(x, rank_table, file_table):
    B, S, D = x.shape
    BB = 256
    grid = (B // BB,)
    return pl.pallas_call(
        _add_body,
        grid=grid,
        in_specs=[
            pl.BlockSpec((BB, S, D), lambda i: (i, 0, 0)),
            pl.BlockSpec((8, 64), lambda i: (0, 0)),
            pl.BlockSpec((8, 64), lambda i: (0, 0)),
        ],
        out_specs=pl.BlockSpec((BB, S, D), lambda i: (i, 0, 0)),
        out_shape=jax.ShapeDtypeStruct((B, S, D), x.dtype),
    )(x, rank_table, file_table)
